# baseline (speedup 1.0000x reference)
import functools

import jax
import jax.numpy as jnp
from jax import lax
from jax.experimental import pallas as pl
from jax.experimental.pallas import tpu as pltpu
from jax.experimental.pallas import tpu_sc as plsc

B = 4096
S = 200
D = 64
HID = 256
NCLS = 50

NC = 2
NS = 16
L = 16
NW = NC * NS
BPW = B // NW
SHALF = S // 2

_mesh = plsc.VectorSubcoreMesh(core_axis_name="c", subcore_axis_name="s")


G = 2
NGRP = BPW // G


@functools.partial(
    pl.kernel,
    out_type=jax.ShapeDtypeStruct((B, D), jnp.float32),
    mesh=_mesh,
    scratch_types=[
        pltpu.VMEM((BPW, 2, SHALF), jnp.int32),
        pltpu.VMEM((G, 2, SHALF, D), jnp.float32),
        pltpu.VMEM((G, 2, SHALF, D), jnp.float32),
        pltpu.VMEM((BPW, D), jnp.float32),
        pltpu.SemaphoreType.DMA,
        pltpu.SemaphoreType.DMA,
    ],
    compiler_params=pltpu.CompilerParams(use_tc_tiling_on_sc=False),
)
def _sc_pool(x_hbm, emb_hbm, out_hbm, idx_v, buf_a, buf_b, pooled_v, sem_a, sem_b):
    wid = lax.axis_index("s") * NC + lax.axis_index("c")
    base = wid * BPW
    pltpu.sync_copy(x_hbm.at[pl.ds(base, BPW)], idx_v)

    def start_group(g, buf, sem):
        for e in range(G):
            i = g * G + e
            for h in range(2):
                pltpu.async_copy(emb_hbm.at[idx_v.at[i, h]], buf.at[e, h], sem)

    def wait_group(g, buf, sem):
        for e in range(G):
            i = g * G + e
            for h in range(2):
                pltpu.make_async_copy(
                    emb_hbm.at[idx_v.at[i, h]], buf.at[e, h], sem
                ).wait()

    RU = 4

    def acc_group(g, buf):
        for e in range(G):
            i = g * G + e

            def body(t, acc, e=e):
                r0 = t * RU
                nxt = list(acc)
                for u in range(RU):
                    for h in range(2):
                        for c in range(4):
                            k = h * 4 + c
                            nxt[k] = nxt[k] + buf[e, h, r0 + u, pl.ds(c * L, L)]
                return tuple(nxt)

            zero = jnp.zeros((L,), jnp.float32)
            acc = lax.fori_loop(0, SHALF // RU, body, (zero,) * 8)
            for c in range(4):
                pooled_v[i, pl.ds(c * L, L)] = acc[c] + acc[4 + c]

    start_group(0, buf_a, sem_a)

    @pl.loop(0, NGRP, step=2)
    def _(g):
        start_group(g + 1, buf_b, sem_b)
        wait_group(g, buf_a, sem_a)
        acc_group(g, buf_a)

        @pl.when(g + 2 < NGRP)
        def _():
            start_group(g + 2, buf_a, sem_a)

        wait_group(g + 1, buf_b, sem_b)
        acc_group(g + 1, buf_b)

    pltpu.sync_copy(pooled_v, out_hbm.at[pl.ds(base, BPW)])


def _mlp_body(p_ref, w1_ref, b1_ref, w2_ref, b2_ref, o_ref):
    p = p_ref[...] * (1.0 / S)
    h = jnp.dot(p, w1_ref[...], preferred_element_type=jnp.float32)
    h = jnp.maximum(h + b1_ref[...], 0.0)
    o_ref[...] = jnp.dot(h, w2_ref[...], preferred_element_type=jnp.float32) + b2_ref[...]


def _mlp(pooled, W1, b1, W2, b2):
    BT = 512
    return pl.pallas_call(
        _mlp_body,
        grid=(B // BT,),
        in_specs=[
            pl.BlockSpec((BT, D), lambda i: (i, 0)),
            pl.BlockSpec((D, HID), lambda i: (0, 0)),
            pl.BlockSpec((1, HID), lambda i: (0, 0)),
            pl.BlockSpec((HID, NCLS), lambda i: (0, 0)),
            pl.BlockSpec((1, NCLS), lambda i: (0, 0)),
        ],
        out_specs=pl.BlockSpec((BT, NCLS), lambda i: (i, 0)),
        out_shape=jax.ShapeDtypeStruct((B, NCLS), jnp.float32),
    )(pooled, W1, b1.reshape(1, HID), W2, b2.reshape(1, NCLS))


def kernel(x, emb, W1, b1, W2, b2):
    x3 = x.astype(jnp.int32).reshape(B, 2, SHALF)
    pooled = _sc_pool(x3, emb)
    return _mlp(pooled, W1, b1, W2, b2)


# speedup vs baseline: 1.0288x; 1.0288x over previous
import functools

import jax
import jax.numpy as jnp
from jax import lax
from jax.experimental import pallas as pl
from jax.experimental.pallas import tpu as pltpu
from jax.experimental.pallas import tpu_sc as plsc

B = 4096
S = 200
D = 64
HID = 256
NCLS = 50

NC = 2
NS = 16
L = 16
NW = NC * NS
BPW = B // NW
SHALF = S // 2

_mesh = plsc.VectorSubcoreMesh(core_axis_name="c", subcore_axis_name="s")


NSLOT = 4


@functools.partial(
    pl.kernel,
    out_type=jax.ShapeDtypeStruct((B, D), jnp.float32),
    mesh=_mesh,
    scratch_types=[
        pltpu.VMEM((BPW, 2, SHALF), jnp.int32),
        pltpu.VMEM((NSLOT, 2, SHALF, D), jnp.float32),
        pltpu.VMEM((BPW, D), jnp.float32),
        pltpu.SemaphoreType.DMA,
        pltpu.SemaphoreType.DMA,
        pltpu.SemaphoreType.DMA,
        pltpu.SemaphoreType.DMA,
    ],
    compiler_params=pltpu.CompilerParams(use_tc_tiling_on_sc=False),
)
def _sc_pool(x_hbm, emb_hbm, out_hbm, idx_v, buf_v, pooled_v, s0, s1, s2, s3):
    sems = (s0, s1, s2, s3)
    wid = lax.axis_index("s") * NC + lax.axis_index("c")
    base = wid * BPW
    pltpu.sync_copy(x_hbm.at[pl.ds(base, BPW)], idx_v)

    def start_elem(i, b):
        for h in range(2):
            pltpu.async_copy(emb_hbm.at[idx_v.at[i, h]], buf_v.at[b, h], sems[b])

    def wait_elem(i, b):
        for h in range(2):
            pltpu.make_async_copy(
                emb_hbm.at[idx_v.at[i, h]], buf_v.at[b, h], sems[b]
            ).wait()

    RU = 4

    def acc_elem(i, b):
        def body(t, acc):
            r0 = t * RU
            nxt = list(acc)
            for u in range(RU):
                for h in range(2):
                    for c in range(4):
                        k = h * 4 + c
                        nxt[k] = nxt[k] + buf_v[b, h, r0 + u, pl.ds(c * L, L)]
            return tuple(nxt)

        zero = jnp.zeros((L,), jnp.float32)
        acc = lax.fori_loop(0, SHALF // RU, body, (zero,) * 8)
        for c in range(4):
            pooled_v[i, pl.ds(c * L, L)] = acc[c] + acc[4 + c]

    for b in range(NSLOT):
        start_elem(b, b)

    @pl.loop(0, BPW, step=NSLOT)
    def _(i):
        for b in range(NSLOT):
            wait_elem(i + b, b)
            acc_elem(i + b, b)

            @pl.when(i + b + NSLOT < BPW)
            def _(b=b):
                start_elem(i + b + NSLOT, b)

    pltpu.sync_copy(pooled_v, out_hbm.at[pl.ds(base, BPW)])


def _mlp_body(p_ref, w1_ref, b1_ref, w2_ref, b2_ref, o_ref):
    p = p_ref[...] * (1.0 / S)
    h = jnp.dot(p, w1_ref[...], preferred_element_type=jnp.float32)
    h = jnp.maximum(h + b1_ref[...], 0.0)
    o_ref[...] = jnp.dot(h, w2_ref[...], preferred_element_type=jnp.float32) + b2_ref[...]


def _mlp(pooled, W1, b1, W2, b2):
    BT = 512
    return pl.pallas_call(
        _mlp_body,
        grid=(B // BT,),
        in_specs=[
            pl.BlockSpec((BT, D), lambda i: (i, 0)),
            pl.BlockSpec((D, HID), lambda i: (0, 0)),
            pl.BlockSpec((1, HID), lambda i: (0, 0)),
            pl.BlockSpec((HID, NCLS), lambda i: (0, 0)),
            pl.BlockSpec((1, NCLS), lambda i: (0, 0)),
        ],
        out_specs=pl.BlockSpec((BT, NCLS), lambda i: (i, 0)),
        out_shape=jax.ShapeDtypeStruct((B, NCLS), jnp.float32),
    )(pooled, W1, b1.reshape(1, HID), W2, b2.reshape(1, NCLS))


def kernel(x, emb, W1, b1, W2, b2):
    x3 = x.astype(jnp.int32).reshape(B, 2, SHALF)
    pooled = _sc_pool(x3, emb)
    return _mlp(pooled, W1, b1, W2, b2)


# speedup vs baseline: 1.0349x; 1.0059x over previous
import functools

import jax
import jax.numpy as jnp
from jax import lax
from jax.experimental import pallas as pl
from jax.experimental.pallas import tpu as pltpu
from jax.experimental.pallas import tpu_sc as plsc

B = 4096
S = 200
D = 64
HID = 256
NCLS = 50

NC = 2
NS = 16
L = 16
NW = NC * NS
BPW = B // NW
SHALF = S // 2

_mesh = plsc.VectorSubcoreMesh(core_axis_name="c", subcore_axis_name="s")


NSLOT = 4


@functools.partial(
    pl.kernel,
    out_type=jax.ShapeDtypeStruct((B, D), jnp.float32),
    mesh=_mesh,
    scratch_types=[
        pltpu.VMEM((BPW, S), jnp.int32),
        pltpu.VMEM((NSLOT, S, D), jnp.float32),
        pltpu.VMEM((BPW, D), jnp.float32),
        pltpu.SemaphoreType.DMA,
        pltpu.SemaphoreType.DMA,
        pltpu.SemaphoreType.DMA,
        pltpu.SemaphoreType.DMA,
    ],
    compiler_params=pltpu.CompilerParams(use_tc_tiling_on_sc=False),
)
def _sc_pool(x_hbm, emb_hbm, out_hbm, idx_v, buf_v, pooled_v, s0, s1, s2, s3):
    sems = (s0, s1, s2, s3)
    wid = lax.axis_index("s") * NC + lax.axis_index("c")
    base = wid * BPW
    pltpu.sync_copy(x_hbm.at[pl.ds(base, BPW)], idx_v)

    def start_elem(i, b):
        pltpu.async_copy(emb_hbm.at[idx_v.at[i]], buf_v.at[b], sems[b])

    def wait_elem(i, b):
        pltpu.make_async_copy(emb_hbm.at[idx_v.at[i]], buf_v.at[b], sems[b]).wait()

    RU = 4

    def acc_elem(i, b):
        def body(t, acc):
            r0 = t * RU * 2
            nxt = list(acc)
            for u in range(RU * 2):
                for c in range(4):
                    k = (u % 2) * 4 + c
                    nxt[k] = nxt[k] + buf_v[b, r0 + u, pl.ds(c * L, L)]
            return tuple(nxt)

        zero = jnp.zeros((L,), jnp.float32)
        acc = lax.fori_loop(0, S // (RU * 2), body, (zero,) * 8)
        for c in range(4):
            pooled_v[i, pl.ds(c * L, L)] = acc[c] + acc[4 + c]

    for b in range(NSLOT):
        start_elem(b, b)

    @pl.loop(0, BPW, step=NSLOT)
    def _(i):
        for b in range(NSLOT):
            wait_elem(i + b, b)
            acc_elem(i + b, b)

            @pl.when(i + b + NSLOT < BPW)
            def _(b=b):
                start_elem(i + b + NSLOT, b)

    pltpu.sync_copy(pooled_v, out_hbm.at[pl.ds(base, BPW)])


def _mlp_body(p_ref, w1_ref, b1_ref, w2_ref, b2_ref, o_ref):
    p = p_ref[...] * (1.0 / S)
    h = jnp.dot(p, w1_ref[...], preferred_element_type=jnp.float32)
    h = jnp.maximum(h + b1_ref[...], 0.0)
    o_ref[...] = jnp.dot(h, w2_ref[...], preferred_element_type=jnp.float32) + b2_ref[...]


def _mlp(pooled, W1, b1, W2, b2):
    BT = 512
    return pl.pallas_call(
        _mlp_body,
        grid=(B // BT,),
        in_specs=[
            pl.BlockSpec((BT, D), lambda i: (i, 0)),
            pl.BlockSpec((D, HID), lambda i: (0, 0)),
            pl.BlockSpec((1, HID), lambda i: (0, 0)),
            pl.BlockSpec((HID, NCLS), lambda i: (0, 0)),
            pl.BlockSpec((1, NCLS), lambda i: (0, 0)),
        ],
        out_specs=pl.BlockSpec((BT, NCLS), lambda i: (i, 0)),
        out_shape=jax.ShapeDtypeStruct((B, NCLS), jnp.float32),
    )(pooled, W1, b1.reshape(1, HID), W2, b2.reshape(1, NCLS))


def kernel(x, emb, W1, b1, W2, b2):
    pooled = _sc_pool(x.astype(jnp.int32), emb)
    return _mlp(pooled, W1, b1, W2, b2)
